# baseline (speedup 1.0000x reference)
import functools

import jax
import jax.numpy as jnp
from jax import lax
from jax.experimental import pallas as pl
from jax.experimental.pallas import tpu as pltpu
from jax.experimental.pallas import tpu_sc as plsc

_N = 10000
_E = 320000
_D = 128
_B = 64
_C = 16

_K = 64
_KD = 128
_NSC = 2
_NTEC = 16
_NW = _NSC * _NTEC
_NPAD = 10240
_STRIPE = _NPAD // _NTEC
_EPAD = 327680
_PER_W = _EPAD // _NW
_NCHUNK_DEG = _PER_W // _KD

_NCHUNK_C = (232, 88)
_NBUF = 4

_BLK = 640
_NBLK = _NPAD // _BLK


@functools.lru_cache(maxsize=None)
def _build_deg_kernel():
    mesh = plsc.VectorSubcoreMesh(core_axis_name="c", subcore_axis_name="s")

    @functools.partial(
        pl.kernel,
        mesh=mesh,
        out_type=jax.ShapeDtypeStruct((_NSC, _NPAD, _D), jnp.float32),
        scratch_types=[
            pltpu.VMEM((_KD,), jnp.int32),
            pltpu.VMEM((_KD, _D), jnp.float32),
            pltpu.VMEM_SHARED((_NPAD, _D), jnp.float32),
        ],
    )
    def deg_kernel(dst_hbm, out_hbm, didx_v, buf_v, acc_sh):
        c = lax.axis_index("c")
        s = lax.axis_index("s")

        def fill(val):
            def body(i, carry):
                for j in range(_D // 16):
                    buf_v[i, pl.ds(j * 16, 16)] = jnp.full((16,), val,
                                                           jnp.float32)
                return carry
            lax.fori_loop(0, _KD, body, 0)

        fill(0.0)
        for blk in range(_STRIPE // _KD):
            pltpu.sync_copy(buf_v,
                            acc_sh.at[pl.ds(s * _STRIPE + blk * _KD, _KD), :])
        fill(1.0)
        plsc.subcore_barrier()

        wid = c * _NTEC + s
        base = wid * _PER_W

        def chunk(j, carry):
            pltpu.sync_copy(dst_hbm.at[pl.ds(base + j * _KD, _KD)], didx_v)
            pltpu.sync_copy(buf_v, acc_sh.at[didx_v], add=True)
            return carry

        lax.fori_loop(0, _NCHUNK_DEG, chunk, 0)
        plsc.subcore_barrier()
        pltpu.sync_copy(
            acc_sh.at[pl.ds(s * _STRIPE, _STRIPE), :],
            out_hbm.at[c, pl.ds(s * _STRIPE, _STRIPE), :],
        )

    return deg_kernel


@functools.lru_cache(maxsize=None)
def _build_edge_kernel():
    mesh = plsc.VectorSubcoreMesh(core_axis_name="c", subcore_axis_name="s")

    @functools.partial(
        pl.kernel,
        mesh=mesh,
        out_type=jax.ShapeDtypeStruct((_NSC, _NPAD, _D), jnp.float32),
        scratch_types=(
            [pltpu.VMEM((_K,), jnp.int32) for _ in range(2 * _NBUF)]
            + [pltpu.VMEM((_K, _D), jnp.float32) for _ in range(_NBUF)]
            + [pltpu.VMEM_SHARED((_NPAD, _D), jnp.float32)]
            + [pltpu.SemaphoreType.DMA for _ in range(_NBUF)]
        ),
    )
    def edge_kernel(table_hbm, src_hbm, dst_hbm, out_hbm, *refs):
        sidx = refs[0:_NBUF]
        didx = refs[_NBUF:2 * _NBUF]
        rows = refs[2 * _NBUF:3 * _NBUF]
        acc_sh = refs[3 * _NBUF]
        sems = refs[3 * _NBUF + 1:]
        c = lax.axis_index("c")
        s = lax.axis_index("s")

        def zrow(i, carry):
            for j in range(_D // 16):
                rows[0][i, pl.ds(j * 16, 16)] = jnp.zeros((16,), jnp.float32)
            return carry
        lax.fori_loop(0, _K, zrow, 0)
        for blk in range(_STRIPE // _K):
            pltpu.sync_copy(rows[0],
                            acc_sh.at[pl.ds(s * _STRIPE + blk * _K, _K), :])
        plsc.subcore_barrier()

        def fire(base, j, b):
            off = base + j * _K
            pltpu.sync_copy(src_hbm.at[pl.ds(off, _K)], sidx[b])
            pltpu.sync_copy(dst_hbm.at[pl.ds(off, _K)], didx[b])
            pltpu.async_copy(table_hbm.at[sidx[b]], rows[b], sems[b])

        def drain(b):
            pltpu.make_async_copy(table_hbm.at[sidx[b]], rows[b],
                                  sems[b]).wait()
            pltpu.sync_copy(rows[b], acc_sh.at[didx[b]], add=True)

        def run_chunks(nchunk, base):
            for b in range(_NBUF):
                fire(base, b, b)

            def group(jj, carry):
                a = jj * _NBUF
                for b in range(_NBUF):
                    drain(b)
                    fire(base, a + _NBUF + b, b)
                return carry

            lax.fori_loop(0, nchunk // _NBUF - 1, group, 0)
            for b in range(_NBUF):
                drain(b)

        n0, n1 = _NCHUNK_C
        core0_edges = _NTEC * n0 * _K

        @pl.when(c == 0)
        def _():
            run_chunks(n0, s * n0 * _K)

        @pl.when(c == 1)
        def _():
            run_chunks(n1, core0_edges + s * n1 * _K)

        plsc.subcore_barrier()
        pltpu.sync_copy(
            acc_sh.at[pl.ds(s * _STRIPE, _STRIPE), :],
            out_hbm.at[c, pl.ds(s * _STRIPE, _STRIPE), :],
        )

    return edge_kernel


def _dinv_block(degp_ref, i):
    deg = degp_ref[0, :, 0:1] + degp_ref[1, :, 0:1] + 1.0
    dinv = lax.rsqrt(jnp.maximum(deg, 1.0))
    row = lax.broadcasted_iota(jnp.int32, (_BLK, 1), 0) + i * _BLK
    dinvm = jnp.where(row < _N, dinv, 0.0)
    return dinv, dinvm


def _t1_body(degp_ref, x_ref, w_ref, out_ref):
    i = pl.program_id(0)
    _, dinvm = _dinv_block(degp_ref, i)
    xw = jnp.dot(x_ref[...], w_ref[...], preferred_element_type=jnp.float32)
    out_ref[...] = xw * dinvm


def _t2_body(degp_ref, p_ref, ys_ref, b_ref, w_ref, out_ref):
    i = pl.program_id(0)
    dinv, dinvm = _dinv_block(degp_ref, i)
    ssum = p_ref[0] + p_ref[1] + ys_ref[...]
    h = jnp.maximum(ssum * dinv + b_ref[...], 0.0)
    out_ref[...] = jnp.dot(h, w_ref[...], preferred_element_type=jnp.float32) * dinvm


def _t3_body(degp_ref, p_ref, ys_ref, b_ref, batch_ref, wfc_ref, bfc_ref,
             out_ref, pooled_acc, cnt_acc):
    i = pl.program_id(0)

    @pl.when(i == 0)
    def _():
        pooled_acc[...] = jnp.zeros_like(pooled_acc)
        cnt_acc[...] = jnp.zeros_like(cnt_acc)

    dinv, _ = _dinv_block(degp_ref, i)
    h3 = (p_ref[0] + p_ref[1] + ys_ref[...]) * dinv + b_ref[...]
    bb = batch_ref[pl.ds(i * _BLK, _BLK)]
    onehot = (bb[None, :] == lax.broadcasted_iota(jnp.int32, (_B, _BLK), 0)
              ).astype(jnp.float32)
    pooled_acc[...] += jnp.dot(onehot, h3, preferred_element_type=jnp.float32)
    cnt_acc[...] += jnp.sum(onehot, axis=1, keepdims=True)

    @pl.when(i == _NBLK - 1)
    def _():
        pooled = pooled_acc[...] / jnp.maximum(cnt_acc[...], 1.0)
        out_ref[...] = (
            jnp.dot(pooled, wfc_ref[...], preferred_element_type=jnp.float32)
            + bfc_ref[...]
        )


def _t1(degp, x_p, W1):
    return pl.pallas_call(
        _t1_body,
        grid=(_NBLK,),
        in_specs=[
            pl.BlockSpec((_NSC, _BLK, _D), lambda i: (0, i, 0)),
            pl.BlockSpec((_BLK, _D), lambda i: (i, 0)),
            pl.BlockSpec((_D, _D), lambda i: (0, 0)),
        ],
        out_specs=pl.BlockSpec((_BLK, _D), lambda i: (i, 0)),
        out_shape=jax.ShapeDtypeStruct((_NPAD, _D), jnp.float32),
    )(degp, x_p, W1)


def _t2(degp, p, ys, b2d, Wn):
    return pl.pallas_call(
        _t2_body,
        grid=(_NBLK,),
        in_specs=[
            pl.BlockSpec((_NSC, _BLK, _D), lambda i: (0, i, 0)),
            pl.BlockSpec((_NSC, _BLK, _D), lambda i: (0, i, 0)),
            pl.BlockSpec((_BLK, _D), lambda i: (i, 0)),
            pl.BlockSpec((1, _D), lambda i: (0, 0)),
            pl.BlockSpec((_D, _D), lambda i: (0, 0)),
        ],
        out_specs=pl.BlockSpec((_BLK, _D), lambda i: (i, 0)),
        out_shape=jax.ShapeDtypeStruct((_NPAD, _D), jnp.float32),
    )(degp, p, ys, b2d, Wn)


def _t3(degp, p, ys, b2d, batch_p, Wfc, bfc2d):
    return pl.pallas_call(
        _t3_body,
        grid=(_NBLK,),
        in_specs=[
            pl.BlockSpec((_NSC, _BLK, _D), lambda i: (0, i, 0)),
            pl.BlockSpec((_NSC, _BLK, _D), lambda i: (0, i, 0)),
            pl.BlockSpec((_BLK, _D), lambda i: (i, 0)),
            pl.BlockSpec((1, _D), lambda i: (0, 0)),
            pl.BlockSpec((_NPAD,), lambda i: (0,)),
            pl.BlockSpec((_D, _C), lambda i: (0, 0)),
            pl.BlockSpec((1, _C), lambda i: (0, 0)),
        ],
        out_specs=pl.BlockSpec((_B, _C), lambda i: (0, 0)),
        out_shape=jax.ShapeDtypeStruct((_B, _C), jnp.float32),
        scratch_shapes=[
            pltpu.VMEM((_B, _D), jnp.float32),
            pltpu.VMEM((_B, 1), jnp.float32),
        ],
    )(degp, p, ys, b2d, batch_p, Wfc, bfc2d)


def kernel(x, edge_index, batch, W1, b1, W2, b2, W3, b3, Wfc, bfc):
    src = edge_index[0]
    dst = edge_index[1]
    pad_e = _EPAD - _E
    fill_src = jnp.full((pad_e,), _N, jnp.int32)
    fill_dst = _N + (jnp.arange(pad_e, dtype=jnp.int32) % (_NPAD - _N))
    src_p = jnp.concatenate([src, fill_src])
    dst_p = jnp.concatenate([dst, fill_dst])
    x_p = jnp.pad(x, ((0, _NPAD - _N), (0, 0)))
    batch_p = jnp.concatenate(
        [batch, jnp.full((_NPAD - _N,), _B, jnp.int32)])

    edge_k = _build_edge_kernel()
    degp = _build_deg_kernel()(dst_p)
    ys1 = _t1(degp, x_p, W1)
    p1 = edge_k(ys1, src_p, dst_p)
    ys2 = _t2(degp, p1, ys1, b1.reshape(1, _D), W2)
    p2 = edge_k(ys2, src_p, dst_p)
    ys3 = _t2(degp, p2, ys2, b2.reshape(1, _D), W3)
    p3 = edge_k(ys3, src_p, dst_p)
    return _t3(degp, p3, ys3, b3.reshape(1, _D), batch_p, Wfc,
               bfc.reshape(1, _C))


# speedup vs baseline: 1.0025x; 1.0025x over previous
import functools

import jax
import jax.numpy as jnp
from jax import lax
from jax.experimental import pallas as pl
from jax.experimental.pallas import tpu as pltpu
from jax.experimental.pallas import tpu_sc as plsc

_N = 10000
_E = 320000
_D = 128
_B = 64
_C = 16

_K = 128
_KD = 128
_NSC = 2
_NTEC = 16
_NW = _NSC * _NTEC
_NPAD = 10240
_STRIPE = _NPAD // _NTEC
_EPAD = 327680
_PER_W = _EPAD // _NW
_NCHUNK_DEG = _PER_W // _KD

_NCHUNK_C = (116, 44)
_NBUF = 2

_BLK = 640
_NBLK = _NPAD // _BLK


@functools.lru_cache(maxsize=None)
def _build_deg_kernel():
    mesh = plsc.VectorSubcoreMesh(core_axis_name="c", subcore_axis_name="s")

    @functools.partial(
        pl.kernel,
        mesh=mesh,
        out_type=jax.ShapeDtypeStruct((_NSC, _NPAD, _D), jnp.float32),
        scratch_types=[
            pltpu.VMEM((_KD,), jnp.int32),
            pltpu.VMEM((_KD, _D), jnp.float32),
            pltpu.VMEM_SHARED((_NPAD, _D), jnp.float32),
        ],
    )
    def deg_kernel(dst_hbm, out_hbm, didx_v, buf_v, acc_sh):
        c = lax.axis_index("c")
        s = lax.axis_index("s")

        def fill(val):
            def body(i, carry):
                for j in range(_D // 16):
                    buf_v[i, pl.ds(j * 16, 16)] = jnp.full((16,), val,
                                                           jnp.float32)
                return carry
            lax.fori_loop(0, _KD, body, 0)

        fill(0.0)
        for blk in range(_STRIPE // _KD):
            pltpu.sync_copy(buf_v,
                            acc_sh.at[pl.ds(s * _STRIPE + blk * _KD, _KD), :])
        fill(1.0)
        plsc.subcore_barrier()

        wid = c * _NTEC + s
        base = wid * _PER_W

        def chunk(j, carry):
            pltpu.sync_copy(dst_hbm.at[pl.ds(base + j * _KD, _KD)], didx_v)
            pltpu.sync_copy(buf_v, acc_sh.at[didx_v], add=True)
            return carry

        lax.fori_loop(0, _NCHUNK_DEG, chunk, 0)
        plsc.subcore_barrier()
        pltpu.sync_copy(
            acc_sh.at[pl.ds(s * _STRIPE, _STRIPE), :],
            out_hbm.at[c, pl.ds(s * _STRIPE, _STRIPE), :],
        )

    return deg_kernel


@functools.lru_cache(maxsize=None)
def _build_edge_kernel():
    mesh = plsc.VectorSubcoreMesh(core_axis_name="c", subcore_axis_name="s")

    @functools.partial(
        pl.kernel,
        mesh=mesh,
        out_type=jax.ShapeDtypeStruct((_NSC, _NPAD, _D), jnp.float32),
        scratch_types=(
            [pltpu.VMEM((_K,), jnp.int32) for _ in range(2 * _NBUF)]
            + [pltpu.VMEM((_K, _D), jnp.float32) for _ in range(_NBUF)]
            + [pltpu.VMEM_SHARED((_NPAD, _D), jnp.float32)]
            + [pltpu.SemaphoreType.DMA for _ in range(_NBUF)]
        ),
    )
    def edge_kernel(table_hbm, src_hbm, dst_hbm, out_hbm, *refs):
        sidx = refs[0:_NBUF]
        didx = refs[_NBUF:2 * _NBUF]
        rows = refs[2 * _NBUF:3 * _NBUF]
        acc_sh = refs[3 * _NBUF]
        sems = refs[3 * _NBUF + 1:]
        c = lax.axis_index("c")
        s = lax.axis_index("s")

        def zrow(i, carry):
            for j in range(_D // 16):
                rows[0][i, pl.ds(j * 16, 16)] = jnp.zeros((16,), jnp.float32)
            return carry
        lax.fori_loop(0, _K, zrow, 0)
        for blk in range(_STRIPE // _K):
            pltpu.sync_copy(rows[0],
                            acc_sh.at[pl.ds(s * _STRIPE + blk * _K, _K), :])
        plsc.subcore_barrier()

        def fire(base, j, b):
            off = base + j * _K
            pltpu.sync_copy(src_hbm.at[pl.ds(off, _K)], sidx[b])
            pltpu.sync_copy(dst_hbm.at[pl.ds(off, _K)], didx[b])
            pltpu.async_copy(table_hbm.at[sidx[b]], rows[b], sems[b])

        def drain(b):
            pltpu.make_async_copy(table_hbm.at[sidx[b]], rows[b],
                                  sems[b]).wait()
            pltpu.sync_copy(rows[b], acc_sh.at[didx[b]], add=True)

        def run_chunks(nchunk, base):
            for b in range(_NBUF):
                fire(base, b, b)

            def group(jj, carry):
                a = jj * _NBUF
                for b in range(_NBUF):
                    drain(b)
                    fire(base, a + _NBUF + b, b)
                return carry

            lax.fori_loop(0, nchunk // _NBUF - 1, group, 0)
            for b in range(_NBUF):
                drain(b)

        n0, n1 = _NCHUNK_C
        core0_edges = _NTEC * n0 * _K

        @pl.when(c == 0)
        def _():
            run_chunks(n0, s * n0 * _K)

        @pl.when(c == 1)
        def _():
            run_chunks(n1, core0_edges + s * n1 * _K)

        plsc.subcore_barrier()
        pltpu.sync_copy(
            acc_sh.at[pl.ds(s * _STRIPE, _STRIPE), :],
            out_hbm.at[c, pl.ds(s * _STRIPE, _STRIPE), :],
        )

    return edge_kernel


def _dinv_block(degp_ref, i):
    deg = degp_ref[0, :, 0:1] + degp_ref[1, :, 0:1] + 1.0
    dinv = lax.rsqrt(jnp.maximum(deg, 1.0))
    row = lax.broadcasted_iota(jnp.int32, (_BLK, 1), 0) + i * _BLK
    dinvm = jnp.where(row < _N, dinv, 0.0)
    return dinv, dinvm


def _t1_body(degp_ref, x_ref, w_ref, out_ref):
    i = pl.program_id(0)
    _, dinvm = _dinv_block(degp_ref, i)
    xw = jnp.dot(x_ref[...], w_ref[...], preferred_element_type=jnp.float32)
    out_ref[...] = xw * dinvm


def _t2_body(degp_ref, p_ref, ys_ref, b_ref, w_ref, out_ref):
    i = pl.program_id(0)
    dinv, dinvm = _dinv_block(degp_ref, i)
    ssum = p_ref[0] + p_ref[1] + ys_ref[...]
    h = jnp.maximum(ssum * dinv + b_ref[...], 0.0)
    out_ref[...] = jnp.dot(h, w_ref[...], preferred_element_type=jnp.float32) * dinvm


def _t3_body(degp_ref, p_ref, ys_ref, b_ref, batch_ref, wfc_ref, bfc_ref,
             out_ref, pooled_acc, cnt_acc):
    i = pl.program_id(0)

    @pl.when(i == 0)
    def _():
        pooled_acc[...] = jnp.zeros_like(pooled_acc)
        cnt_acc[...] = jnp.zeros_like(cnt_acc)

    dinv, _ = _dinv_block(degp_ref, i)
    h3 = (p_ref[0] + p_ref[1] + ys_ref[...]) * dinv + b_ref[...]
    bb = batch_ref[pl.ds(i * _BLK, _BLK)]
    onehot = (bb[None, :] == lax.broadcasted_iota(jnp.int32, (_B, _BLK), 0)
              ).astype(jnp.float32)
    pooled_acc[...] += jnp.dot(onehot, h3, preferred_element_type=jnp.float32)
    cnt_acc[...] += jnp.sum(onehot, axis=1, keepdims=True)

    @pl.when(i == _NBLK - 1)
    def _():
        pooled = pooled_acc[...] / jnp.maximum(cnt_acc[...], 1.0)
        out_ref[...] = (
            jnp.dot(pooled, wfc_ref[...], preferred_element_type=jnp.float32)
            + bfc_ref[...]
        )


def _t1(degp, x_p, W1):
    return pl.pallas_call(
        _t1_body,
        grid=(_NBLK,),
        in_specs=[
            pl.BlockSpec((_NSC, _BLK, _D), lambda i: (0, i, 0)),
            pl.BlockSpec((_BLK, _D), lambda i: (i, 0)),
            pl.BlockSpec((_D, _D), lambda i: (0, 0)),
        ],
        out_specs=pl.BlockSpec((_BLK, _D), lambda i: (i, 0)),
        out_shape=jax.ShapeDtypeStruct((_NPAD, _D), jnp.float32),
    )(degp, x_p, W1)


def _t2(degp, p, ys, b2d, Wn):
    return pl.pallas_call(
        _t2_body,
        grid=(_NBLK,),
        in_specs=[
            pl.BlockSpec((_NSC, _BLK, _D), lambda i: (0, i, 0)),
            pl.BlockSpec((_NSC, _BLK, _D), lambda i: (0, i, 0)),
            pl.BlockSpec((_BLK, _D), lambda i: (i, 0)),
            pl.BlockSpec((1, _D), lambda i: (0, 0)),
            pl.BlockSpec((_D, _D), lambda i: (0, 0)),
        ],
        out_specs=pl.BlockSpec((_BLK, _D), lambda i: (i, 0)),
        out_shape=jax.ShapeDtypeStruct((_NPAD, _D), jnp.float32),
    )(degp, p, ys, b2d, Wn)


def _t3(degp, p, ys, b2d, batch_p, Wfc, bfc2d):
    return pl.pallas_call(
        _t3_body,
        grid=(_NBLK,),
        in_specs=[
            pl.BlockSpec((_NSC, _BLK, _D), lambda i: (0, i, 0)),
            pl.BlockSpec((_NSC, _BLK, _D), lambda i: (0, i, 0)),
            pl.BlockSpec((_BLK, _D), lambda i: (i, 0)),
            pl.BlockSpec((1, _D), lambda i: (0, 0)),
            pl.BlockSpec((_NPAD,), lambda i: (0,)),
            pl.BlockSpec((_D, _C), lambda i: (0, 0)),
            pl.BlockSpec((1, _C), lambda i: (0, 0)),
        ],
        out_specs=pl.BlockSpec((_B, _C), lambda i: (0, 0)),
        out_shape=jax.ShapeDtypeStruct((_B, _C), jnp.float32),
        scratch_shapes=[
            pltpu.VMEM((_B, _D), jnp.float32),
            pltpu.VMEM((_B, 1), jnp.float32),
        ],
    )(degp, p, ys, b2d, batch_p, Wfc, bfc2d)


def kernel(x, edge_index, batch, W1, b1, W2, b2, W3, b3, Wfc, bfc):
    src = edge_index[0]
    dst = edge_index[1]
    pad_e = _EPAD - _E
    fill_src = jnp.full((pad_e,), _N, jnp.int32)
    fill_dst = _N + (jnp.arange(pad_e, dtype=jnp.int32) % (_NPAD - _N))
    src_p = jnp.concatenate([src, fill_src])
    dst_p = jnp.concatenate([dst, fill_dst])
    x_p = jnp.pad(x, ((0, _NPAD - _N), (0, 0)))
    batch_p = jnp.concatenate(
        [batch, jnp.full((_NPAD - _N,), _B, jnp.int32)])

    edge_k = _build_edge_kernel()
    degp = _build_deg_kernel()(dst_p)
    ys1 = _t1(degp, x_p, W1)
    p1 = edge_k(ys1, src_p, dst_p)
    ys2 = _t2(degp, p1, ys1, b1.reshape(1, _D), W2)
    p2 = edge_k(ys2, src_p, dst_p)
    ys3 = _t2(degp, p2, ys2, b2.reshape(1, _D), W3)
    p3 = edge_k(ys3, src_p, dst_p)
    return _t3(degp, p3, ys3, b3.reshape(1, _D), batch_p, Wfc,
               bfc.reshape(1, _C))


# speedup vs baseline: 2.3684x; 2.3625x over previous
import functools

import jax
import jax.numpy as jnp
from jax import lax
from jax.experimental import pallas as pl
from jax.experimental.pallas import tpu as pltpu
from jax.experimental.pallas import tpu_sc as plsc

_N = 10000
_E = 320000
_D = 128
_B = 64
_C = 16

_K = 128
_G = 4
_NSC = 2
_NTEC = 16
_NW = _NSC * _NTEC
_NPAD = 10240
_STRIPE = _NPAD // _NTEC
_EPAD = 327680
_PER_W = _EPAD // _NW
_NCHUNK_DEG = _PER_W // _K

_NCHUNK_C = (120, 40)

_BLK = 640
_NBLK = _NPAD // _BLK


@functools.lru_cache(maxsize=None)
def _build_deg_kernel():
    mesh = plsc.VectorSubcoreMesh(core_axis_name="c", subcore_axis_name="s")

    @functools.partial(
        pl.kernel,
        mesh=mesh,
        out_type=jax.ShapeDtypeStruct((_NSC, _NPAD, _D), jnp.float32),
        scratch_types=[
            pltpu.VMEM((_G, _K), jnp.int32),
            pltpu.VMEM((_G, _K), jnp.int32),
            pltpu.VMEM((_K, _D), jnp.float32),
            pltpu.VMEM_SHARED((_NPAD, _D), jnp.float32),
            pltpu.SemaphoreType.DMA,
            pltpu.SemaphoreType.DMA,
        ],
    )
    def deg_kernel(dst2d_hbm, out_hbm, dblk0, dblk1, buf_v, acc_sh,
                   isem0, isem1):
        dblk = (dblk0, dblk1)
        isem = (isem0, isem1)
        c = lax.axis_index("c")
        s = lax.axis_index("s")

        def fill(val):
            def body(i, carry):
                for j in range(_D // 16):
                    buf_v[i, pl.ds(j * 16, 16)] = jnp.full((16,), val,
                                                           jnp.float32)
                return carry
            lax.fori_loop(0, _K, body, 0)

        fill(0.0)
        for blk in range(_STRIPE // _K):
            pltpu.sync_copy(buf_v,
                            acc_sh.at[pl.ds(s * _STRIPE + blk * _K, _K), :])
        fill(1.0)
        plsc.subcore_barrier()

        wid = c * _NTEC + s
        base_row = wid * (_PER_W // _K)

        def load_blk(row, p):
            pltpu.async_copy(dst2d_hbm.at[pl.ds(row, _G), :], dblk[p],
                             isem[p])

        def wait_blk(row, p):
            pltpu.make_async_copy(dst2d_hbm.at[pl.ds(row, _G), :], dblk[p],
                                  isem[p]).wait()

        def group(row, p, load_row):
            wait_blk(row, p)
            for m in range(_G):
                pltpu.sync_copy(buf_v, acc_sh.at[dblk[p].at[m]], add=True)
            if load_row is not None:
                load_blk(load_row, p)

        ng = _NCHUNK_DEG // _G
        load_blk(base_row, 0)
        load_blk(base_row + _G, 1)

        def pairs(g2, carry):
            arow = base_row + (2 * g2) * _G
            group(arow, 0, arow + 2 * _G)
            group(arow + _G, 1, arow + 3 * _G)
            return carry

        lax.fori_loop(0, ng // 2 - 1, pairs, 0)
        group(base_row + (ng - 2) * _G, 0, None)
        group(base_row + (ng - 1) * _G, 1, None)

        plsc.subcore_barrier()
        pltpu.sync_copy(
            acc_sh.at[pl.ds(s * _STRIPE, _STRIPE), :],
            out_hbm.at[c, pl.ds(s * _STRIPE, _STRIPE), :],
        )

    return deg_kernel


@functools.lru_cache(maxsize=None)
def _build_edge_kernel():
    mesh = plsc.VectorSubcoreMesh(core_axis_name="c", subcore_axis_name="s")

    @functools.partial(
        pl.kernel,
        mesh=mesh,
        out_type=jax.ShapeDtypeStruct((_NSC, _NPAD, _D), jnp.float32),
        scratch_types=(
            [pltpu.VMEM((_G, _K), jnp.int32) for _ in range(4)]
            + [pltpu.VMEM((_K, _D), jnp.float32) for _ in range(2)]
            + [pltpu.VMEM_SHARED((_NPAD, _D), jnp.float32)]
            + [pltpu.SemaphoreType.DMA for _ in range(4)]
        ),
    )
    def edge_kernel(table_hbm, src2d_hbm, dst2d_hbm, out_hbm, *refs):
        sblk = refs[0:2]
        dblk = refs[2:4]
        rows = refs[4:6]
        acc_sh = refs[6]
        isem = refs[7:9]
        rsem = refs[9:11]
        c = lax.axis_index("c")
        s = lax.axis_index("s")

        def zrow(i, carry):
            for j in range(_D // 16):
                rows[0][i, pl.ds(j * 16, 16)] = jnp.zeros((16,), jnp.float32)
            return carry
        lax.fori_loop(0, _K, zrow, 0)
        for blk in range(_STRIPE // _K):
            pltpu.sync_copy(rows[0],
                            acc_sh.at[pl.ds(s * _STRIPE + blk * _K, _K), :])
        plsc.subcore_barrier()

        def load_blk(row, p):
            pltpu.async_copy(src2d_hbm.at[pl.ds(row, _G), :], sblk[p],
                             isem[p])
            pltpu.async_copy(dst2d_hbm.at[pl.ds(row, _G), :], dblk[p],
                             isem[p])

        def wait_blk(row, p):
            pltpu.make_async_copy(src2d_hbm.at[pl.ds(row, _G), :], sblk[p],
                                  isem[p]).wait()
            pltpu.make_async_copy(dst2d_hbm.at[pl.ds(row, _G), :], dblk[p],
                                  isem[p]).wait()

        def fire(p, m, b):
            pltpu.async_copy(table_hbm.at[sblk[p].at[m]], rows[b], rsem[b])

        def drain(p, m, b):
            pltpu.make_async_copy(table_hbm.at[sblk[p].at[m]], rows[b],
                                  rsem[b]).wait()
            pltpu.sync_copy(rows[b], acc_sh.at[dblk[p].at[m]], add=True)

        def group(p, np_, next_row, load_row):
            for m in range(_G - 1):
                fire(p, m + 1, (m + 1) % 2)
                drain(p, m, m % 2)
            if next_row is not None:
                wait_blk(next_row, np_)
                fire(np_, 0, 0)
            drain(p, _G - 1, (_G - 1) % 2)
            if load_row is not None:
                load_blk(load_row, p)

        def run_chunks(nchunk, base_row):
            ng = nchunk // _G
            load_blk(base_row, 0)
            wait_blk(base_row, 0)
            load_blk(base_row + _G, 1)
            fire(0, 0, 0)

            def pairs(g2, carry):
                arow = base_row + (2 * g2) * _G
                group(0, 1, arow + _G, arow + 2 * _G)
                group(1, 0, arow + 2 * _G, arow + 3 * _G)
                return carry

            lax.fori_loop(0, ng // 2 - 1, pairs, 0)
            group(0, 1, base_row + (ng - 1) * _G, None)
            group(1, 0, None, None)

        n0, n1 = _NCHUNK_C
        core0_rows = _NTEC * n0

        @pl.when(c == 0)
        def _():
            run_chunks(n0, s * n0)

        @pl.when(c == 1)
        def _():
            run_chunks(n1, core0_rows + s * n1)

        plsc.subcore_barrier()
        pltpu.sync_copy(
            acc_sh.at[pl.ds(s * _STRIPE, _STRIPE), :],
            out_hbm.at[c, pl.ds(s * _STRIPE, _STRIPE), :],
        )

    return edge_kernel


def _dinv_block(degp_ref, i):
    deg = degp_ref[0, :, 0:1] + degp_ref[1, :, 0:1] + 1.0
    dinv = lax.rsqrt(jnp.maximum(deg, 1.0))
    row = lax.broadcasted_iota(jnp.int32, (_BLK, 1), 0) + i * _BLK
    dinvm = jnp.where(row < _N, dinv, 0.0)
    return dinv, dinvm


def _t1_body(degp_ref, x_ref, w_ref, out_ref):
    i = pl.program_id(0)
    _, dinvm = _dinv_block(degp_ref, i)
    xw = jnp.dot(x_ref[...], w_ref[...], preferred_element_type=jnp.float32)
    out_ref[...] = xw * dinvm


def _t2_body(degp_ref, p_ref, ys_ref, b_ref, w_ref, out_ref):
    i = pl.program_id(0)
    dinv, dinvm = _dinv_block(degp_ref, i)
    ssum = p_ref[0] + p_ref[1] + ys_ref[...]
    h = jnp.maximum(ssum * dinv + b_ref[...], 0.0)
    out_ref[...] = jnp.dot(h, w_ref[...], preferred_element_type=jnp.float32) * dinvm


def _t3_body(degp_ref, p_ref, ys_ref, b_ref, batch_ref, wfc_ref, bfc_ref,
             out_ref, pooled_acc, cnt_acc):
    i = pl.program_id(0)

    @pl.when(i == 0)
    def _():
        pooled_acc[...] = jnp.zeros_like(pooled_acc)
        cnt_acc[...] = jnp.zeros_like(cnt_acc)

    dinv, _ = _dinv_block(degp_ref, i)
    h3 = (p_ref[0] + p_ref[1] + ys_ref[...]) * dinv + b_ref[...]
    bb = batch_ref[pl.ds(i * _BLK, _BLK)]
    onehot = (bb[None, :] == lax.broadcasted_iota(jnp.int32, (_B, _BLK), 0)
              ).astype(jnp.float32)
    pooled_acc[...] += jnp.dot(onehot, h3, preferred_element_type=jnp.float32)
    cnt_acc[...] += jnp.sum(onehot, axis=1, keepdims=True)

    @pl.when(i == _NBLK - 1)
    def _():
        pooled = pooled_acc[...] / jnp.maximum(cnt_acc[...], 1.0)
        out_ref[...] = (
            jnp.dot(pooled, wfc_ref[...], preferred_element_type=jnp.float32)
            + bfc_ref[...]
        )


def _t1(degp, x_p, W1):
    return pl.pallas_call(
        _t1_body,
        grid=(_NBLK,),
        in_specs=[
            pl.BlockSpec((_NSC, _BLK, _D), lambda i: (0, i, 0)),
            pl.BlockSpec((_BLK, _D), lambda i: (i, 0)),
            pl.BlockSpec((_D, _D), lambda i: (0, 0)),
        ],
        out_specs=pl.BlockSpec((_BLK, _D), lambda i: (i, 0)),
        out_shape=jax.ShapeDtypeStruct((_NPAD, _D), jnp.float32),
    )(degp, x_p, W1)


def _t2(degp, p, ys, b2d, Wn):
    return pl.pallas_call(
        _t2_body,
        grid=(_NBLK,),
        in_specs=[
            pl.BlockSpec((_NSC, _BLK, _D), lambda i: (0, i, 0)),
            pl.BlockSpec((_NSC, _BLK, _D), lambda i: (0, i, 0)),
            pl.BlockSpec((_BLK, _D), lambda i: (i, 0)),
            pl.BlockSpec((1, _D), lambda i: (0, 0)),
            pl.BlockSpec((_D, _D), lambda i: (0, 0)),
        ],
        out_specs=pl.BlockSpec((_BLK, _D), lambda i: (i, 0)),
        out_shape=jax.ShapeDtypeStruct((_NPAD, _D), jnp.float32),
    )(degp, p, ys, b2d, Wn)


def _t3(degp, p, ys, b2d, batch_p, Wfc, bfc2d):
    return pl.pallas_call(
        _t3_body,
        grid=(_NBLK,),
        in_specs=[
            pl.BlockSpec((_NSC, _BLK, _D), lambda i: (0, i, 0)),
            pl.BlockSpec((_NSC, _BLK, _D), lambda i: (0, i, 0)),
            pl.BlockSpec((_BLK, _D), lambda i: (i, 0)),
            pl.BlockSpec((1, _D), lambda i: (0, 0)),
            pl.BlockSpec((_NPAD,), lambda i: (0,)),
            pl.BlockSpec((_D, _C), lambda i: (0, 0)),
            pl.BlockSpec((1, _C), lambda i: (0, 0)),
        ],
        out_specs=pl.BlockSpec((_B, _C), lambda i: (0, 0)),
        out_shape=jax.ShapeDtypeStruct((_B, _C), jnp.float32),
        scratch_shapes=[
            pltpu.VMEM((_B, _D), jnp.float32),
            pltpu.VMEM((_B, 1), jnp.float32),
        ],
    )(degp, p, ys, b2d, batch_p, Wfc, bfc2d)


def kernel(x, edge_index, batch, W1, b1, W2, b2, W3, b3, Wfc, bfc):
    src = edge_index[0]
    dst = edge_index[1]
    pad_e = _EPAD - _E
    fill = _N + (jnp.arange(pad_e, dtype=jnp.int32) % (_NPAD - _N))
    src2d = jnp.concatenate([src, fill]).reshape(_EPAD // _K, _K)
    dst2d = jnp.concatenate([dst, fill]).reshape(_EPAD // _K, _K)
    x_p = jnp.pad(x, ((0, _NPAD - _N), (0, 0)))
    batch_p = jnp.concatenate(
        [batch, jnp.full((_NPAD - _N,), _B, jnp.int32)])

    edge_k = _build_edge_kernel()
    degp = _build_deg_kernel()(dst2d)
    ys1 = _t1(degp, x_p, W1)
    p1 = edge_k(ys1, src2d, dst2d)
    ys2 = _t2(degp, p1, ys1, b1.reshape(1, _D), W2)
    p2 = edge_k(ys2, src2d, dst2d)
    ys3 = _t2(degp, p2, ys2, b2.reshape(1, _D), W3)
    p3 = edge_k(ys3, src2d, dst2d)
    return _t3(degp, p3, ys3, b3.reshape(1, _D), batch_p, Wfc,
               bfc.reshape(1, _C))


# speedup vs baseline: 2.8617x; 1.2083x over previous
import functools

import jax
import jax.numpy as jnp
from jax import lax
from jax.experimental import pallas as pl
from jax.experimental.pallas import tpu as pltpu
from jax.experimental.pallas import tpu_sc as plsc

_N = 10000
_E = 320000
_D = 128
_B = 64
_C = 16

_K = 128
_G = 4
_NSC = 2
_NTEC = 16
_NW = _NSC * _NTEC
_NPAD = 10240
_STRIPE = _NPAD // _NTEC
_EPAD = 327680
_PER_W = _EPAD // _NW
_NCHUNK_DEG = _PER_W // _K

_NCHUNK_C = (88, 72)

_BLK = 640
_NBLK = _NPAD // _BLK


@functools.lru_cache(maxsize=None)
def _build_deg_kernel():
    mesh = plsc.VectorSubcoreMesh(core_axis_name="c", subcore_axis_name="s")

    @functools.partial(
        pl.kernel,
        mesh=mesh,
        out_type=jax.ShapeDtypeStruct((_NSC, _NPAD, _D), jnp.float32),
        scratch_types=[
            pltpu.VMEM((_G, _K), jnp.int32),
            pltpu.VMEM((_G, _K), jnp.int32),
            pltpu.VMEM((_K, _D), jnp.float32),
            pltpu.VMEM_SHARED((_NPAD, _D), jnp.float32),
            pltpu.SemaphoreType.DMA,
            pltpu.SemaphoreType.DMA,
        ],
    )
    def deg_kernel(dst2d_hbm, out_hbm, dblk0, dblk1, buf_v, acc_sh,
                   isem0, isem1):
        dblk = (dblk0, dblk1)
        isem = (isem0, isem1)
        c = lax.axis_index("c")
        s = lax.axis_index("s")

        def fill(val):
            def body(i, carry):
                for j in range(_D // 16):
                    buf_v[i, pl.ds(j * 16, 16)] = jnp.full((16,), val,
                                                           jnp.float32)
                return carry
            lax.fori_loop(0, _K, body, 0)

        fill(0.0)
        for blk in range(_STRIPE // _K):
            pltpu.sync_copy(buf_v,
                            acc_sh.at[pl.ds(s * _STRIPE + blk * _K, _K), :])
        fill(1.0)
        plsc.subcore_barrier()

        wid = c * _NTEC + s
        base_row = wid * (_PER_W // _K)

        def load_blk(row, p):
            pltpu.async_copy(dst2d_hbm.at[pl.ds(row, _G), :], dblk[p],
                             isem[p])

        def wait_blk(row, p):
            pltpu.make_async_copy(dst2d_hbm.at[pl.ds(row, _G), :], dblk[p],
                                  isem[p]).wait()

        def group(row, p, load_row):
            wait_blk(row, p)
            for m in range(_G):
                pltpu.sync_copy(buf_v, acc_sh.at[dblk[p].at[m]], add=True)
            if load_row is not None:
                load_blk(load_row, p)

        ng = _NCHUNK_DEG // _G
        load_blk(base_row, 0)
        load_blk(base_row + _G, 1)

        def pairs(g2, carry):
            arow = base_row + (2 * g2) * _G
            group(arow, 0, arow + 2 * _G)
            group(arow + _G, 1, arow + 3 * _G)
            return carry

        lax.fori_loop(0, ng // 2 - 1, pairs, 0)
        group(base_row + (ng - 2) * _G, 0, None)
        group(base_row + (ng - 1) * _G, 1, None)

        plsc.subcore_barrier()
        pltpu.sync_copy(
            acc_sh.at[pl.ds(s * _STRIPE, _STRIPE), :],
            out_hbm.at[c, pl.ds(s * _STRIPE, _STRIPE), :],
        )

    return deg_kernel


@functools.lru_cache(maxsize=None)
def _build_edge_kernel():
    mesh = plsc.VectorSubcoreMesh(core_axis_name="c", subcore_axis_name="s")

    @functools.partial(
        pl.kernel,
        mesh=mesh,
        out_type=jax.ShapeDtypeStruct((_NSC, _NPAD, _D), jnp.float32),
        scratch_types=(
            [pltpu.VMEM((_G, _K), jnp.int32) for _ in range(4)]
            + [pltpu.VMEM((_K, _D), jnp.float32) for _ in range(2)]
            + [pltpu.VMEM_SHARED((_NPAD, _D), jnp.float32)]
            + [pltpu.SemaphoreType.DMA for _ in range(4)]
        ),
    )
    def edge_kernel(table_hbm, src2d_hbm, dst2d_hbm, out_hbm, *refs):
        sblk = refs[0:2]
        dblk = refs[2:4]
        rows = refs[4:6]
        acc_sh = refs[6]
        isem = refs[7:9]
        rsem = refs[9:11]
        c = lax.axis_index("c")
        s = lax.axis_index("s")

        def zrow(i, carry):
            for j in range(_D // 16):
                rows[0][i, pl.ds(j * 16, 16)] = jnp.zeros((16,), jnp.float32)
            return carry
        lax.fori_loop(0, _K, zrow, 0)
        for blk in range(_STRIPE // _K):
            pltpu.sync_copy(rows[0],
                            acc_sh.at[pl.ds(s * _STRIPE + blk * _K, _K), :])
        plsc.subcore_barrier()

        def load_blk(row, p):
            pltpu.async_copy(src2d_hbm.at[pl.ds(row, _G), :], sblk[p],
                             isem[p])
            pltpu.async_copy(dst2d_hbm.at[pl.ds(row, _G), :], dblk[p],
                             isem[p])

        def wait_blk(row, p):
            pltpu.make_async_copy(src2d_hbm.at[pl.ds(row, _G), :], sblk[p],
                                  isem[p]).wait()
            pltpu.make_async_copy(dst2d_hbm.at[pl.ds(row, _G), :], dblk[p],
                                  isem[p]).wait()

        def fire(p, m, b):
            pltpu.async_copy(table_hbm.at[sblk[p].at[m]], rows[b], rsem[b])

        def drain(p, m, b):
            pltpu.make_async_copy(table_hbm.at[sblk[p].at[m]], rows[b],
                                  rsem[b]).wait()
            pltpu.sync_copy(rows[b], acc_sh.at[dblk[p].at[m]], add=True)

        def group(p, np_, next_row, load_row):
            for m in range(_G - 1):
                fire(p, m + 1, (m + 1) % 2)
                drain(p, m, m % 2)
            if next_row is not None:
                wait_blk(next_row, np_)
                fire(np_, 0, 0)
            drain(p, _G - 1, (_G - 1) % 2)
            if load_row is not None:
                load_blk(load_row, p)

        def run_chunks(nchunk, base_row):
            ng = nchunk // _G
            load_blk(base_row, 0)
            wait_blk(base_row, 0)
            load_blk(base_row + _G, 1)
            fire(0, 0, 0)

            def pairs(g2, carry):
                arow = base_row + (2 * g2) * _G
                group(0, 1, arow + _G, arow + 2 * _G)
                group(1, 0, arow + 2 * _G, arow + 3 * _G)
                return carry

            lax.fori_loop(0, ng // 2 - 1, pairs, 0)
            group(0, 1, base_row + (ng - 1) * _G, None)
            group(1, 0, None, None)

        n0, n1 = _NCHUNK_C
        core0_rows = _NTEC * n0

        @pl.when(c == 0)
        def _():
            run_chunks(n0, s * n0)

        @pl.when(c == 1)
        def _():
            run_chunks(n1, core0_rows + s * n1)

        plsc.subcore_barrier()
        pltpu.sync_copy(
            acc_sh.at[pl.ds(s * _STRIPE, _STRIPE), :],
            out_hbm.at[c, pl.ds(s * _STRIPE, _STRIPE), :],
        )

    return edge_kernel


def _dinv_block(degp_ref, i):
    deg = degp_ref[0, :, 0:1] + degp_ref[1, :, 0:1] + 1.0
    dinv = lax.rsqrt(jnp.maximum(deg, 1.0))
    row = lax.broadcasted_iota(jnp.int32, (_BLK, 1), 0) + i * _BLK
    dinvm = jnp.where(row < _N, dinv, 0.0)
    return dinv, dinvm


def _t1_body(degp_ref, x_ref, w_ref, out_ref):
    i = pl.program_id(0)
    _, dinvm = _dinv_block(degp_ref, i)
    xw = jnp.dot(x_ref[...], w_ref[...], preferred_element_type=jnp.float32)
    out_ref[...] = xw * dinvm


def _t2_body(degp_ref, p_ref, ys_ref, b_ref, w_ref, out_ref):
    i = pl.program_id(0)
    dinv, dinvm = _dinv_block(degp_ref, i)
    ssum = p_ref[0] + p_ref[1] + ys_ref[...]
    h = jnp.maximum(ssum * dinv + b_ref[...], 0.0)
    out_ref[...] = jnp.dot(h, w_ref[...], preferred_element_type=jnp.float32) * dinvm


def _t3_body(degp_ref, p_ref, ys_ref, b_ref, batch_ref, wfc_ref, bfc_ref,
             out_ref, pooled_acc, cnt_acc):
    i = pl.program_id(0)

    @pl.when(i == 0)
    def _():
        pooled_acc[...] = jnp.zeros_like(pooled_acc)
        cnt_acc[...] = jnp.zeros_like(cnt_acc)

    dinv, _ = _dinv_block(degp_ref, i)
    h3 = (p_ref[0] + p_ref[1] + ys_ref[...]) * dinv + b_ref[...]
    bb = batch_ref[pl.ds(i * _BLK, _BLK)]
    onehot = (bb[None, :] == lax.broadcasted_iota(jnp.int32, (_B, _BLK), 0)
              ).astype(jnp.float32)
    pooled_acc[...] += jnp.dot(onehot, h3, preferred_element_type=jnp.float32)
    cnt_acc[...] += jnp.sum(onehot, axis=1, keepdims=True)

    @pl.when(i == _NBLK - 1)
    def _():
        pooled = pooled_acc[...] / jnp.maximum(cnt_acc[...], 1.0)
        out_ref[...] = (
            jnp.dot(pooled, wfc_ref[...], preferred_element_type=jnp.float32)
            + bfc_ref[...]
        )


def _t1(degp, x_p, W1):
    return pl.pallas_call(
        _t1_body,
        grid=(_NBLK,),
        in_specs=[
            pl.BlockSpec((_NSC, _BLK, _D), lambda i: (0, i, 0)),
            pl.BlockSpec((_BLK, _D), lambda i: (i, 0)),
            pl.BlockSpec((_D, _D), lambda i: (0, 0)),
        ],
        out_specs=pl.BlockSpec((_BLK, _D), lambda i: (i, 0)),
        out_shape=jax.ShapeDtypeStruct((_NPAD, _D), jnp.float32),
    )(degp, x_p, W1)


def _t2(degp, p, ys, b2d, Wn):
    return pl.pallas_call(
        _t2_body,
        grid=(_NBLK,),
        in_specs=[
            pl.BlockSpec((_NSC, _BLK, _D), lambda i: (0, i, 0)),
            pl.BlockSpec((_NSC, _BLK, _D), lambda i: (0, i, 0)),
            pl.BlockSpec((_BLK, _D), lambda i: (i, 0)),
            pl.BlockSpec((1, _D), lambda i: (0, 0)),
            pl.BlockSpec((_D, _D), lambda i: (0, 0)),
        ],
        out_specs=pl.BlockSpec((_BLK, _D), lambda i: (i, 0)),
        out_shape=jax.ShapeDtypeStruct((_NPAD, _D), jnp.float32),
    )(degp, p, ys, b2d, Wn)


def _t3(degp, p, ys, b2d, batch_p, Wfc, bfc2d):
    return pl.pallas_call(
        _t3_body,
        grid=(_NBLK,),
        in_specs=[
            pl.BlockSpec((_NSC, _BLK, _D), lambda i: (0, i, 0)),
            pl.BlockSpec((_NSC, _BLK, _D), lambda i: (0, i, 0)),
            pl.BlockSpec((_BLK, _D), lambda i: (i, 0)),
            pl.BlockSpec((1, _D), lambda i: (0, 0)),
            pl.BlockSpec((_NPAD,), lambda i: (0,)),
            pl.BlockSpec((_D, _C), lambda i: (0, 0)),
            pl.BlockSpec((1, _C), lambda i: (0, 0)),
        ],
        out_specs=pl.BlockSpec((_B, _C), lambda i: (0, 0)),
        out_shape=jax.ShapeDtypeStruct((_B, _C), jnp.float32),
        scratch_shapes=[
            pltpu.VMEM((_B, _D), jnp.float32),
            pltpu.VMEM((_B, 1), jnp.float32),
        ],
    )(degp, p, ys, b2d, batch_p, Wfc, bfc2d)


def kernel(x, edge_index, batch, W1, b1, W2, b2, W3, b3, Wfc, bfc):
    src = edge_index[0]
    dst = edge_index[1]
    pad_e = _EPAD - _E
    fill = _N + (jnp.arange(pad_e, dtype=jnp.int32) % (_NPAD - _N))
    src2d = jnp.concatenate([src, fill]).reshape(_EPAD // _K, _K)
    dst2d = jnp.concatenate([dst, fill]).reshape(_EPAD // _K, _K)
    x_p = jnp.pad(x, ((0, _NPAD - _N), (0, 0)))
    batch_p = jnp.concatenate(
        [batch, jnp.full((_NPAD - _N,), _B, jnp.int32)])

    edge_k = _build_edge_kernel()
    degp = _build_deg_kernel()(dst2d)
    ys1 = _t1(degp, x_p, W1)
    p1 = edge_k(ys1, src2d, dst2d)
    ys2 = _t2(degp, p1, ys1, b1.reshape(1, _D), W2)
    p2 = edge_k(ys2, src2d, dst2d)
    ys3 = _t2(degp, p2, ys2, b2.reshape(1, _D), W3)
    p3 = edge_k(ys3, src2d, dst2d)
    return _t3(degp, p3, ys3, b3.reshape(1, _D), batch_p, Wfc,
               bfc.reshape(1, _C))


# speedup vs baseline: 3.0085x; 1.0513x over previous
import functools

import jax
import jax.numpy as jnp
from jax import lax
from jax.experimental import pallas as pl
from jax.experimental.pallas import tpu as pltpu
from jax.experimental.pallas import tpu_sc as plsc

_N = 10000
_E = 320000
_D = 128
_B = 64
_C = 16

_K = 128
_G = 4
_NSC = 2
_NTEC = 16
_NW = _NSC * _NTEC
_NPAD = 10240
_STRIPE = _NPAD // _NTEC
_EPAD = 327680
_PER_W = _EPAD // _NW
_NCHUNK_DEG = _PER_W // _K

_NCHUNK_C = (80, 80)

_BLK = 640
_NBLK = _NPAD // _BLK


@functools.lru_cache(maxsize=None)
def _build_deg_kernel():
    mesh = plsc.VectorSubcoreMesh(core_axis_name="c", subcore_axis_name="s")

    @functools.partial(
        pl.kernel,
        mesh=mesh,
        out_type=jax.ShapeDtypeStruct((_NSC, _NPAD, _D), jnp.float32),
        scratch_types=[
            pltpu.VMEM((_G, _K), jnp.int32),
            pltpu.VMEM((_G, _K), jnp.int32),
            pltpu.VMEM((_K, _D), jnp.float32),
            pltpu.VMEM_SHARED((_NPAD, _D), jnp.float32),
            pltpu.SemaphoreType.DMA,
            pltpu.SemaphoreType.DMA,
        ],
    )
    def deg_kernel(dst2d_hbm, out_hbm, dblk0, dblk1, buf_v, acc_sh,
                   isem0, isem1):
        dblk = (dblk0, dblk1)
        isem = (isem0, isem1)
        c = lax.axis_index("c")
        s = lax.axis_index("s")

        def fill(val):
            def body(i, carry):
                for j in range(_D // 16):
                    buf_v[i, pl.ds(j * 16, 16)] = jnp.full((16,), val,
                                                           jnp.float32)
                return carry
            lax.fori_loop(0, _K, body, 0)

        fill(0.0)
        for blk in range(_STRIPE // _K):
            pltpu.sync_copy(buf_v,
                            acc_sh.at[pl.ds(s * _STRIPE + blk * _K, _K), :])
        fill(1.0)
        plsc.subcore_barrier()

        wid = c * _NTEC + s
        base_row = wid * (_PER_W // _K)

        def load_blk(row, p):
            pltpu.async_copy(dst2d_hbm.at[pl.ds(row, _G), :], dblk[p],
                             isem[p])

        def wait_blk(row, p):
            pltpu.make_async_copy(dst2d_hbm.at[pl.ds(row, _G), :], dblk[p],
                                  isem[p]).wait()

        def group(row, p, load_row):
            wait_blk(row, p)
            for m in range(_G):
                pltpu.sync_copy(buf_v, acc_sh.at[dblk[p].at[m]], add=True)
            if load_row is not None:
                load_blk(load_row, p)

        ng = _NCHUNK_DEG // _G
        load_blk(base_row, 0)
        load_blk(base_row + _G, 1)

        def pairs(g2, carry):
            arow = base_row + (2 * g2) * _G
            group(arow, 0, arow + 2 * _G)
            group(arow + _G, 1, arow + 3 * _G)
            return carry

        lax.fori_loop(0, ng // 2 - 1, pairs, 0)
        group(base_row + (ng - 2) * _G, 0, None)
        group(base_row + (ng - 1) * _G, 1, None)

        plsc.subcore_barrier()
        pltpu.sync_copy(
            acc_sh.at[pl.ds(s * _STRIPE, _STRIPE), :],
            out_hbm.at[c, pl.ds(s * _STRIPE, _STRIPE), :],
        )

    return deg_kernel


@functools.lru_cache(maxsize=None)
def _build_edge_kernel():
    mesh = plsc.VectorSubcoreMesh(core_axis_name="c", subcore_axis_name="s")

    @functools.partial(
        pl.kernel,
        mesh=mesh,
        out_type=jax.ShapeDtypeStruct((_NSC, _NPAD, _D), jnp.float32),
        scratch_types=(
            [pltpu.VMEM((_G, _K), jnp.int32) for _ in range(4)]
            + [pltpu.VMEM((_K, _D), jnp.float32) for _ in range(2)]
            + [pltpu.VMEM_SHARED((_NPAD, _D), jnp.float32)]
            + [pltpu.SemaphoreType.DMA for _ in range(4)]
        ),
    )
    def edge_kernel(table_hbm, src2d_hbm, dst2d_hbm, out_hbm, *refs):
        sblk = refs[0:2]
        dblk = refs[2:4]
        rows = refs[4:6]
        acc_sh = refs[6]
        isem = refs[7:9]
        rsem = refs[9:11]
        c = lax.axis_index("c")
        s = lax.axis_index("s")

        def zrow(i, carry):
            for j in range(_D // 16):
                rows[0][i, pl.ds(j * 16, 16)] = jnp.zeros((16,), jnp.float32)
            return carry
        lax.fori_loop(0, _K, zrow, 0)
        for blk in range(_STRIPE // _K):
            pltpu.sync_copy(rows[0],
                            acc_sh.at[pl.ds(s * _STRIPE + blk * _K, _K), :])
        plsc.subcore_barrier()

        def load_blk(row, p):
            pltpu.async_copy(src2d_hbm.at[pl.ds(row, _G), :], sblk[p],
                             isem[p])
            pltpu.async_copy(dst2d_hbm.at[pl.ds(row, _G), :], dblk[p],
                             isem[p])

        def wait_blk(row, p):
            pltpu.make_async_copy(src2d_hbm.at[pl.ds(row, _G), :], sblk[p],
                                  isem[p]).wait()
            pltpu.make_async_copy(dst2d_hbm.at[pl.ds(row, _G), :], dblk[p],
                                  isem[p]).wait()

        def fire(p, m, b):
            pltpu.async_copy(table_hbm.at[sblk[p].at[m]], rows[b], rsem[b])

        def drain(p, m, b):
            pltpu.make_async_copy(table_hbm.at[sblk[p].at[m]], rows[b],
                                  rsem[b]).wait()
            pltpu.sync_copy(rows[b], acc_sh.at[dblk[p].at[m]], add=True)

        def group(p, np_, next_row, load_row):
            for m in range(_G - 1):
                fire(p, m + 1, (m + 1) % 2)
                drain(p, m, m % 2)
            if next_row is not None:
                wait_blk(next_row, np_)
                fire(np_, 0, 0)
            drain(p, _G - 1, (_G - 1) % 2)
            if load_row is not None:
                load_blk(load_row, p)

        def run_chunks(nchunk, base_row):
            ng = nchunk // _G
            load_blk(base_row, 0)
            wait_blk(base_row, 0)
            load_blk(base_row + _G, 1)
            fire(0, 0, 0)

            def pairs(g2, carry):
                arow = base_row + (2 * g2) * _G
                group(0, 1, arow + _G, arow + 2 * _G)
                group(1, 0, arow + 2 * _G, arow + 3 * _G)
                return carry

            lax.fori_loop(0, ng // 2 - 1, pairs, 0)
            group(0, 1, base_row + (ng - 1) * _G, None)
            group(1, 0, None, None)

        n0, n1 = _NCHUNK_C
        core0_rows = _NTEC * n0

        @pl.when(c == 0)
        def _():
            run_chunks(n0, s * n0)

        @pl.when(c == 1)
        def _():
            run_chunks(n1, core0_rows + s * n1)

        plsc.subcore_barrier()
        pltpu.sync_copy(
            acc_sh.at[pl.ds(s * _STRIPE, _STRIPE), :],
            out_hbm.at[c, pl.ds(s * _STRIPE, _STRIPE), :],
        )

    return edge_kernel


def _dinv_block(degp_ref, i):
    deg = degp_ref[0, :, 0:1] + degp_ref[1, :, 0:1] + 1.0
    dinv = lax.rsqrt(jnp.maximum(deg, 1.0))
    row = lax.broadcasted_iota(jnp.int32, (_BLK, 1), 0) + i * _BLK
    dinvm = jnp.where(row < _N, dinv, 0.0)
    return dinv, dinvm


def _t1_body(degp_ref, x_ref, w_ref, out_ref):
    i = pl.program_id(0)
    _, dinvm = _dinv_block(degp_ref, i)
    xw = jnp.dot(x_ref[...], w_ref[...], preferred_element_type=jnp.float32)
    out_ref[...] = xw * dinvm


def _t2_body(degp_ref, p_ref, ys_ref, b_ref, w_ref, out_ref):
    i = pl.program_id(0)
    dinv, dinvm = _dinv_block(degp_ref, i)
    ssum = p_ref[0] + p_ref[1] + ys_ref[...]
    h = jnp.maximum(ssum * dinv + b_ref[...], 0.0)
    out_ref[...] = jnp.dot(h, w_ref[...], preferred_element_type=jnp.float32) * dinvm


def _t3_body(degp_ref, p_ref, ys_ref, b_ref, batch_ref, wfc_ref, bfc_ref,
             out_ref, pooled_acc, cnt_acc):
    i = pl.program_id(0)

    @pl.when(i == 0)
    def _():
        pooled_acc[...] = jnp.zeros_like(pooled_acc)
        cnt_acc[...] = jnp.zeros_like(cnt_acc)

    dinv, _ = _dinv_block(degp_ref, i)
    h3 = (p_ref[0] + p_ref[1] + ys_ref[...]) * dinv + b_ref[...]
    bb = batch_ref[pl.ds(i * _BLK, _BLK)]
    onehot = (bb[None, :] == lax.broadcasted_iota(jnp.int32, (_B, _BLK), 0)
              ).astype(jnp.float32)
    pooled_acc[...] += jnp.dot(onehot, h3, preferred_element_type=jnp.float32)
    cnt_acc[...] += jnp.sum(onehot, axis=1, keepdims=True)

    @pl.when(i == _NBLK - 1)
    def _():
        pooled = pooled_acc[...] / jnp.maximum(cnt_acc[...], 1.0)
        out_ref[...] = (
            jnp.dot(pooled, wfc_ref[...], preferred_element_type=jnp.float32)
            + bfc_ref[...]
        )


def _t1(degp, x_p, W1):
    return pl.pallas_call(
        _t1_body,
        grid=(_NBLK,),
        in_specs=[
            pl.BlockSpec((_NSC, _BLK, _D), lambda i: (0, i, 0)),
            pl.BlockSpec((_BLK, _D), lambda i: (i, 0)),
            pl.BlockSpec((_D, _D), lambda i: (0, 0)),
        ],
        out_specs=pl.BlockSpec((_BLK, _D), lambda i: (i, 0)),
        out_shape=jax.ShapeDtypeStruct((_NPAD, _D), jnp.float32),
    )(degp, x_p, W1)


def _t2(degp, p, ys, b2d, Wn):
    return pl.pallas_call(
        _t2_body,
        grid=(_NBLK,),
        in_specs=[
            pl.BlockSpec((_NSC, _BLK, _D), lambda i: (0, i, 0)),
            pl.BlockSpec((_NSC, _BLK, _D), lambda i: (0, i, 0)),
            pl.BlockSpec((_BLK, _D), lambda i: (i, 0)),
            pl.BlockSpec((1, _D), lambda i: (0, 0)),
            pl.BlockSpec((_D, _D), lambda i: (0, 0)),
        ],
        out_specs=pl.BlockSpec((_BLK, _D), lambda i: (i, 0)),
        out_shape=jax.ShapeDtypeStruct((_NPAD, _D), jnp.float32),
    )(degp, p, ys, b2d, Wn)


def _t3(degp, p, ys, b2d, batch_p, Wfc, bfc2d):
    return pl.pallas_call(
        _t3_body,
        grid=(_NBLK,),
        in_specs=[
            pl.BlockSpec((_NSC, _BLK, _D), lambda i: (0, i, 0)),
            pl.BlockSpec((_NSC, _BLK, _D), lambda i: (0, i, 0)),
            pl.BlockSpec((_BLK, _D), lambda i: (i, 0)),
            pl.BlockSpec((1, _D), lambda i: (0, 0)),
            pl.BlockSpec((_NPAD,), lambda i: (0,)),
            pl.BlockSpec((_D, _C), lambda i: (0, 0)),
            pl.BlockSpec((1, _C), lambda i: (0, 0)),
        ],
        out_specs=pl.BlockSpec((_B, _C), lambda i: (0, 0)),
        out_shape=jax.ShapeDtypeStruct((_B, _C), jnp.float32),
        scratch_shapes=[
            pltpu.VMEM((_B, _D), jnp.float32),
            pltpu.VMEM((_B, 1), jnp.float32),
        ],
    )(degp, p, ys, b2d, batch_p, Wfc, bfc2d)


def kernel(x, edge_index, batch, W1, b1, W2, b2, W3, b3, Wfc, bfc):
    src = edge_index[0]
    dst = edge_index[1]
    pad_e = _EPAD - _E
    fill = _N + (jnp.arange(pad_e, dtype=jnp.int32) % (_NPAD - _N))
    src2d = jnp.concatenate([src, fill]).reshape(_EPAD // _K, _K)
    dst2d = jnp.concatenate([dst, fill]).reshape(_EPAD // _K, _K)
    x_p = jnp.pad(x, ((0, _NPAD - _N), (0, 0)))
    batch_p = jnp.concatenate(
        [batch, jnp.full((_NPAD - _N,), _B, jnp.int32)])

    edge_k = _build_edge_kernel()
    degp = _build_deg_kernel()(dst2d)
    ys1 = _t1(degp, x_p, W1)
    p1 = edge_k(ys1, src2d, dst2d)
    ys2 = _t2(degp, p1, ys1, b1.reshape(1, _D), W2)
    p2 = edge_k(ys2, src2d, dst2d)
    ys3 = _t2(degp, p2, ys2, b2.reshape(1, _D), W3)
    p3 = edge_k(ys3, src2d, dst2d)
    return _t3(degp, p3, ys3, b3.reshape(1, _D), batch_p, Wfc,
               bfc.reshape(1, _C))
